# baseline (speedup 1.0000x reference)
import jax
import jax.numpy as jnp
from jax.experimental import pallas as pl

_TILE = 2048


def _mlp_kernel(x_ref, w1_ref, b1_ref, w2_ref, b2_ref,
                wa1_ref, ba1_ref, wa2_ref, ba2_ref,
                wc1_ref, bc1_ref, wc2_ref, bc2_ref,
                logits_ref, value_ref):
    x = x_ref[...]
    h = jnp.maximum(
        jnp.dot(x, w1_ref[...], preferred_element_type=jnp.float32) + b1_ref[...], 0.0)
    e = jnp.maximum(
        jnp.dot(h, w2_ref[...], preferred_element_type=jnp.float32) + b2_ref[...], 0.0)
    a = jnp.maximum(
        jnp.dot(e, wa1_ref[...], preferred_element_type=jnp.float32) + ba1_ref[...], 0.0)
    logits_ref[...] = (
        jnp.dot(a, wa2_ref[...], preferred_element_type=jnp.float32) + ba2_ref[...])
    c = jnp.maximum(
        jnp.dot(e, wc1_ref[...], preferred_element_type=jnp.float32) + bc1_ref[...], 0.0)
    value_ref[...] = (
        jnp.dot(c, wc2_ref[...], preferred_element_type=jnp.float32) + bc2_ref[...])


def kernel(global_state, W1, b1, W2, b2, Wa1, ba1, Wa2, ba2, Wc1, bc1, Wc2, bc2):
    B, in_dim = global_state.shape
    n_act = Wa2.shape[1]
    grid = (B // _TILE,)

    def row_block(n):
        return pl.BlockSpec((_TILE, n), lambda i: (i, 0))

    def whole(a):
        return pl.BlockSpec(a.shape, lambda i: (0,) * a.ndim)

    b1r, b2r = b1[None, :], b2[None, :]
    ba1r, ba2r = ba1[None, :], ba2[None, :]
    bc1r, bc2r = bc1[None, :], bc2[None, :]

    logits, value = pl.pallas_call(
        _mlp_kernel,
        grid=grid,
        in_specs=[
            row_block(in_dim),
            whole(W1), whole(b1r), whole(W2), whole(b2r),
            whole(Wa1), whole(ba1r), whole(Wa2), whole(ba2r),
            whole(Wc1), whole(bc1r), whole(Wc2), whole(bc2r),
        ],
        out_specs=[row_block(n_act), row_block(1)],
        out_shape=[
            jax.ShapeDtypeStruct((B, n_act), jnp.float32),
            jax.ShapeDtypeStruct((B, 1), jnp.float32),
        ],
    )(global_state, W1, b1r, W2, b2r, Wa1, ba1r, Wa2, ba2r, Wc1, bc1r, Wc2, bc2r)
    return (logits, value)
